# baseline (speedup 1.0000x reference)
import jax
import jax.numpy as jnp
from jax.experimental import pallas as pl
from jax.experimental.pallas import tpu as pltpu

VOCAB = 100000
EMBED_DIM = 128
CTX_LEN = 200

_NS = 2
_BV = 7168
_NPS = 7


def _tc_matvec_body(idx_sref, emb_ref, *refs):
    w_refs = refs[:_NS]
    b_refs = refs[_NS:2 * _NS]
    out_ref = refs[2 * _NS]
    mean_ref = refs[2 * _NS + 1]
    rows_ref = refs[2 * _NS + 2]
    sem = refs[2 * _NS + 3]

    @pl.when(pl.program_id(0) == 0)
    def _():
        def issue(j, carry):
            base = j * 8
            for u in range(8):
                pltpu.make_async_copy(
                    emb_ref.at[pl.ds(idx_sref[base + u], 1)],
                    rows_ref.at[pl.ds(base + u, 1)],
                    sem,
                ).start()
            return carry

        jax.lax.fori_loop(0, CTX_LEN // 8, issue, 0)
        pltpu.make_async_copy(
            emb_ref.at[pl.ds(0, CTX_LEN)], rows_ref, sem
        ).wait()
        m = jnp.sum(rows_ref[...], axis=0, keepdims=True) * (1.0 / CTX_LEN)
        mean_ref[...] = m

    m = mean_ref[...]
    accs = [
        jax.lax.dot_general(
            m, w_refs[s][...],
            (((1,), (1,)), ((), ())),
            preferred_element_type=jnp.float32,
        ) + b_refs[s][...]
        for s in range(_NS)
    ]
    out_ref[...] = jnp.concatenate(accs, axis=0)


def kernel(context_words, embeddings, W, b):
    b2d = b.reshape(1, VOCAB)
    w_specs = [
        pl.BlockSpec((_BV, EMBED_DIM), lambda i, idx, s=s: (s * _NPS + i, 0))
        for s in range(_NS)
    ]
    b_specs = [
        pl.BlockSpec((1, _BV), lambda i, idx, s=s: (0, s * _NPS + i))
        for s in range(_NS)
    ]
    grid_spec = pltpu.PrefetchScalarGridSpec(
        num_scalar_prefetch=1,
        grid=(_NPS,),
        in_specs=[pl.BlockSpec(memory_space=pltpu.MemorySpace.HBM)] + w_specs + b_specs,
        out_specs=pl.BlockSpec((_NS, _BV), lambda i, idx: (0, i)),
        scratch_shapes=[
            pltpu.VMEM((1, EMBED_DIM), jnp.float32),
            pltpu.VMEM((CTX_LEN, EMBED_DIM), jnp.float32),
            pltpu.SemaphoreType.DMA,
        ],
    )
    out = pl.pallas_call(
        _tc_matvec_body,
        grid_spec=grid_spec,
        out_shape=jax.ShapeDtypeStruct((_NS, _NPS * _BV), jnp.float32),
    )(context_words, embeddings, *([W] * _NS), *([b2d] * _NS))
    return out.reshape(_NS * _NPS * _BV)[:VOCAB]
